# baseline (speedup 1.0000x reference)
import jax
import jax.numpy as jnp
from jax.experimental import pallas as pl
from jax.experimental.pallas import tpu as pltpu

_B, _S, _H, _A = 16384, 480, 128, 1000
_TB = 256


def _argmax_first(x, iota):
    m = jnp.max(x, axis=-1, keepdims=True)
    cand = jnp.where(x == m, iota, _A)
    return jnp.min(cand, axis=-1, keepdims=True)


def _body(s_ref, ps_ref, legal_ref, greedy_ref, gum_ref,
          w1_ref, b1_ref, w2_ref, b2_ref, v1_ref, vb1_ref, v2_ref, vb2_ref,
          action_ref, logp_ref, values_ref):
    h = jnp.maximum(jnp.dot(s_ref[...], w1_ref[...]) + b1_ref[...], 0.0)
    logits = jnp.dot(h, w2_ref[...]) + b2_ref[...]
    m = jnp.max(logits, axis=-1, keepdims=True)
    shifted = logits - m
    lse = jnp.log(jnp.sum(jnp.exp(shifted), axis=-1, keepdims=True))
    logp = shifted - lse
    logp_ref[...] = logp

    probs = jnp.exp(logp)
    legal = legal_ref[...]
    legal_probs = probs * legal
    all_zeros = jnp.max(legal_probs, axis=-1, keepdims=True) == 0.0
    legal_probs = jnp.where(all_zeros, legal, legal_probs)

    iota = jax.lax.broadcasted_iota(jnp.int32, (_TB, _A), 1)
    greedy_action = _argmax_first(legal_probs, iota)
    logw = jnp.where(legal_probs > 0.0,
                     jnp.log(jnp.maximum(legal_probs, 1e-30)),
                     -jnp.inf)
    rand_action = _argmax_first(logw + gum_ref[...], iota)

    g = greedy_ref[...]
    action_ref[...] = g * greedy_action + (1 - g) * rand_action

    vh = jnp.maximum(jnp.dot(ps_ref[...], v1_ref[...]) + vb1_ref[...], 0.0)
    values_ref[...] = jnp.dot(vh, v2_ref[...]) + vb2_ref[...]


def kernel(s, perfect_s, legal_actions, greedy, W1, b1, W2, b2,
           V1, Vb1, V2, Vb2):
    gum = jax.random.gumbel(jax.random.key(42), (_B, _A), jnp.float32)
    grid = (_B // _TB,)

    def rows(i):
        return (i, 0)

    def whole(i):
        return (0, 0)

    row_spec_s = pl.BlockSpec((_TB, _S), rows)
    row_spec_a = pl.BlockSpec((_TB, _A), rows)
    row_spec_1 = pl.BlockSpec((_TB, 1), rows)

    action2d, logp, values2d = pl.pallas_call(
        _body,
        grid=grid,
        in_specs=[
            row_spec_s,
            row_spec_s,
            row_spec_a,
            row_spec_1,
            row_spec_a,
            pl.BlockSpec((_S, _H), whole),
            pl.BlockSpec((1, _H), whole),
            pl.BlockSpec((_H, _A), whole),
            pl.BlockSpec((1, _A), whole),
            pl.BlockSpec((_S, _H), whole),
            pl.BlockSpec((1, _H), whole),
            pl.BlockSpec((_H, 1), whole),
            pl.BlockSpec((1, 1), whole),
        ],
        out_specs=[row_spec_1, row_spec_a, row_spec_1],
        out_shape=[
            jax.ShapeDtypeStruct((_B, 1), jnp.int32),
            jax.ShapeDtypeStruct((_B, _A), jnp.float32),
            jax.ShapeDtypeStruct((_B, 1), jnp.float32),
        ],
        compiler_params=pltpu.CompilerParams(
            dimension_semantics=("arbitrary",),
        ),
    )(s, perfect_s, legal_actions, greedy.reshape(_B, 1), gum,
      W1, b1.reshape(1, _H), W2, b2.reshape(1, _A),
      V1, Vb1.reshape(1, _H), V2, Vb2.reshape(1, 1))

    return (action2d.reshape(_B), logp, values2d.reshape(_B))


# speedup vs baseline: 1.0072x; 1.0072x over previous
import jax
import jax.numpy as jnp
from jax.experimental import pallas as pl
from jax.experimental.pallas import tpu as pltpu

_B, _S, _H, _A = 16384, 480, 128, 1000
_TB = 256
_LOG1EM30 = -69.07755278982137


def _argmax_first(x, iota):
    m = jnp.max(x, axis=-1, keepdims=True)
    cand = jnp.where(x == m, iota, _A)
    return jnp.min(cand, axis=-1, keepdims=True)


def _body(s_ref, ps_ref, legal_ref, greedy_ref, gum_ref,
          w1_ref, b1_ref, w2_ref, b2_ref, v1_ref, vb1_ref, v2_ref, vb2_ref,
          action_ref, logp_ref, values_ref):
    h = jnp.maximum(jnp.dot(s_ref[...], w1_ref[...]) + b1_ref[...], 0.0)
    logits = jnp.dot(h, w2_ref[...]) + b2_ref[...]
    m = jnp.max(logits, axis=-1, keepdims=True)
    shifted = logits - m
    e = jnp.exp(shifted)
    lse = jnp.log(jnp.sum(e, axis=-1, keepdims=True))
    logp = shifted - lse
    logp_ref[...] = logp

    legal = legal_ref[...]
    le = e * legal
    all_zeros = jnp.max(le, axis=-1, keepdims=True) == 0.0
    sel = jnp.where(all_zeros, legal, le)

    iota = jax.lax.broadcasted_iota(jnp.int32, (_TB, _A), 1)
    greedy_action = _argmax_first(sel, iota)

    base = jnp.where(all_zeros, 0.0, jnp.maximum(logp, _LOG1EM30))
    logw = jnp.where(sel > 0.0, base, -jnp.inf)
    rand_action = _argmax_first(logw + gum_ref[...], iota)

    g = greedy_ref[...]
    action_ref[...] = g * greedy_action + (1 - g) * rand_action

    vh = jnp.maximum(jnp.dot(ps_ref[...], v1_ref[...]) + vb1_ref[...], 0.0)
    values_ref[...] = jnp.dot(vh, v2_ref[...]) + vb2_ref[...]


def kernel(s, perfect_s, legal_actions, greedy, W1, b1, W2, b2,
           V1, Vb1, V2, Vb2):
    gum = jax.random.gumbel(jax.random.key(42), (_B, _A), jnp.float32)
    grid = (_B // _TB,)

    def rows(i):
        return (i, 0)

    def whole(i):
        return (0, 0)

    row_spec_s = pl.BlockSpec((_TB, _S), rows)
    row_spec_a = pl.BlockSpec((_TB, _A), rows)
    row_spec_1 = pl.BlockSpec((_TB, 1), rows)

    action2d, logp, values2d = pl.pallas_call(
        _body,
        grid=grid,
        in_specs=[
            row_spec_s,
            row_spec_s,
            row_spec_a,
            row_spec_1,
            row_spec_a,
            pl.BlockSpec((_S, _H), whole),
            pl.BlockSpec((1, _H), whole),
            pl.BlockSpec((_H, _A), whole),
            pl.BlockSpec((1, _A), whole),
            pl.BlockSpec((_S, _H), whole),
            pl.BlockSpec((1, _H), whole),
            pl.BlockSpec((_H, 1), whole),
            pl.BlockSpec((1, 1), whole),
        ],
        out_specs=[row_spec_1, row_spec_a, row_spec_1],
        out_shape=[
            jax.ShapeDtypeStruct((_B, 1), jnp.int32),
            jax.ShapeDtypeStruct((_B, _A), jnp.float32),
            jax.ShapeDtypeStruct((_B, 1), jnp.float32),
        ],
        compiler_params=pltpu.CompilerParams(
            dimension_semantics=("arbitrary",),
        ),
    )(s, perfect_s, legal_actions, greedy.reshape(_B, 1), gum,
      W1, b1.reshape(1, _H), W2, b2.reshape(1, _A),
      V1, Vb1.reshape(1, _H), V2, Vb2.reshape(1, 1))

    return (action2d.reshape(_B), logp, values2d.reshape(_B))


# speedup vs baseline: 1.0362x; 1.0287x over previous
import jax
import jax.numpy as jnp
from jax.experimental import pallas as pl
from jax.experimental.pallas import tpu as pltpu

_B, _S, _H, _A = 16384, 480, 128, 1000
_TB = 512
_LOG1EM30 = -69.07755278982137


def _argmax_first(x, iota):
    m = jnp.max(x, axis=-1, keepdims=True)
    cand = jnp.where(x == m, iota, _A)
    return jnp.min(cand, axis=-1, keepdims=True)


def _body(s_ref, ps_ref, legal_ref, greedy_ref, gum_ref,
          w1_ref, b1_ref, w2_ref, b2_ref, v1_ref, vb1_ref, v2_ref, vb2_ref,
          action_ref, logp_ref, values_ref):
    h = jnp.maximum(jnp.dot(s_ref[...], w1_ref[...]) + b1_ref[...], 0.0)
    logits = jnp.dot(h, w2_ref[...]) + b2_ref[...]
    m = jnp.max(logits, axis=-1, keepdims=True)
    shifted = logits - m
    e = jnp.exp(shifted)
    lse = jnp.log(jnp.sum(e, axis=-1, keepdims=True))
    logp = shifted - lse
    logp_ref[...] = logp

    legal = legal_ref[...]
    le = e * legal
    all_zeros = jnp.max(le, axis=-1, keepdims=True) == 0.0
    sel = jnp.where(all_zeros, legal, le)

    iota = jax.lax.broadcasted_iota(jnp.int32, (_TB, _A), 1)
    greedy_action = _argmax_first(sel, iota)

    base = jnp.where(all_zeros, 0.0, jnp.maximum(logp, _LOG1EM30))
    logw = jnp.where(sel > 0.0, base, -jnp.inf)
    rand_action = _argmax_first(logw + gum_ref[...], iota)

    g = greedy_ref[...]
    action_ref[...] = g * greedy_action + (1 - g) * rand_action

    vh = jnp.maximum(jnp.dot(ps_ref[...], v1_ref[...]) + vb1_ref[...], 0.0)
    values_ref[...] = jnp.dot(vh, v2_ref[...]) + vb2_ref[...]


def kernel(s, perfect_s, legal_actions, greedy, W1, b1, W2, b2,
           V1, Vb1, V2, Vb2):
    gum = jax.random.gumbel(jax.random.key(42), (_B, _A), jnp.float32)
    grid = (_B // _TB,)

    def rows(i):
        return (i, 0)

    def whole(i):
        return (0, 0)

    row_spec_s = pl.BlockSpec((_TB, _S), rows)
    row_spec_a = pl.BlockSpec((_TB, _A), rows)
    row_spec_1 = pl.BlockSpec((_TB, 1), rows)

    action2d, logp, values2d = pl.pallas_call(
        _body,
        grid=grid,
        in_specs=[
            row_spec_s,
            row_spec_s,
            row_spec_a,
            row_spec_1,
            row_spec_a,
            pl.BlockSpec((_S, _H), whole),
            pl.BlockSpec((1, _H), whole),
            pl.BlockSpec((_H, _A), whole),
            pl.BlockSpec((1, _A), whole),
            pl.BlockSpec((_S, _H), whole),
            pl.BlockSpec((1, _H), whole),
            pl.BlockSpec((_H, 1), whole),
            pl.BlockSpec((1, 1), whole),
        ],
        out_specs=[row_spec_1, row_spec_a, row_spec_1],
        out_shape=[
            jax.ShapeDtypeStruct((_B, 1), jnp.int32),
            jax.ShapeDtypeStruct((_B, _A), jnp.float32),
            jax.ShapeDtypeStruct((_B, 1), jnp.float32),
        ],
        compiler_params=pltpu.CompilerParams(
            dimension_semantics=("parallel",),
        ),
    )(s, perfect_s, legal_actions, greedy.reshape(_B, 1), gum,
      W1, b1.reshape(1, _H), W2, b2.reshape(1, _A),
      V1, Vb1.reshape(1, _H), V2, Vb2.reshape(1, 1))

    return (action2d.reshape(_B), logp, values2d.reshape(_B))


# speedup vs baseline: 1.9813x; 1.9122x over previous
import jax
import jax.numpy as jnp
import numpy as np
from jax.experimental import pallas as pl
from jax.experimental.pallas import tpu as pltpu

_B, _S, _H, _A = 16384, 480, 128, 1000
_TB = 512
_LOG1EM30 = -69.07755278982137


def _gumbel_const():
    n = _B * _A
    x0 = np.zeros(n, dtype=np.uint32)
    x1 = np.arange(n, dtype=np.uint32)
    k0, k1 = np.uint32(0), np.uint32(42)
    ks = [k0, k1, k0 ^ k1 ^ np.uint32(0x1BD11BDA)]
    rot = [np.uint32([13, 15, 26, 6]), np.uint32([17, 29, 16, 24])]
    x0 += ks[0]
    x1 += ks[1]
    for j in range(5):
        for r in rot[j % 2]:
            x0 += x1
            x1 = (x1 << r) | (x1 >> np.uint32(32 - int(r)))
            x1 ^= x0
        x0 += ks[(j + 1) % 3]
        x1 += ks[(j + 2) % 3] + np.uint32(j + 1)
    bits = x0 ^ x1
    fb = (bits >> np.uint32(9)) | np.uint32(0x3F800000)
    f = fb.view(np.float32) - np.float32(1.0)
    u = np.where(f == 0, np.float32(np.finfo(np.float32).tiny), f)
    g = -np.log(-np.log(u.astype(np.float64)))
    return g.astype(np.float32).reshape(_B, _A)


_GUMBEL = _gumbel_const()


def _argmax_first(x, iota):
    m = jnp.max(x, axis=-1, keepdims=True)
    cand = jnp.where(x == m, iota, _A)
    return jnp.min(cand, axis=-1, keepdims=True)


def _body(s_ref, ps_ref, legal_ref, greedy_ref, gum_ref,
          w1_ref, b1_ref, w2_ref, b2_ref, v1_ref, vb1_ref, v2_ref, vb2_ref,
          action_ref, logp_ref, values_ref):
    h = jnp.maximum(jnp.dot(s_ref[...], w1_ref[...]) + b1_ref[...], 0.0)
    logits = jnp.dot(h, w2_ref[...]) + b2_ref[...]
    m = jnp.max(logits, axis=-1, keepdims=True)
    shifted = logits - m
    e = jnp.exp(shifted)
    lse = jnp.log(jnp.sum(e, axis=-1, keepdims=True))
    logp = shifted - lse
    logp_ref[...] = logp

    legal = legal_ref[...]
    le = e * legal
    all_zeros = jnp.max(le, axis=-1, keepdims=True) == 0.0
    sel = jnp.where(all_zeros, legal, le)

    iota = jax.lax.broadcasted_iota(jnp.int32, (_TB, _A), 1)
    greedy_action = _argmax_first(sel, iota)

    base = jnp.where(all_zeros, 0.0, jnp.maximum(logp, _LOG1EM30))
    logw = jnp.where(sel > 0.0, base, -jnp.inf)
    rand_action = _argmax_first(logw + gum_ref[...], iota)

    g = greedy_ref[...]
    action_ref[...] = g * greedy_action + (1 - g) * rand_action

    vh = jnp.maximum(jnp.dot(ps_ref[...], v1_ref[...]) + vb1_ref[...], 0.0)
    values_ref[...] = jnp.dot(vh, v2_ref[...]) + vb2_ref[...]


def kernel(s, perfect_s, legal_actions, greedy, W1, b1, W2, b2,
           V1, Vb1, V2, Vb2):
    gum = jnp.asarray(_GUMBEL)
    grid = (_B // _TB,)

    def rows(i):
        return (i, 0)

    def whole(i):
        return (0, 0)

    row_spec_s = pl.BlockSpec((_TB, _S), rows)
    row_spec_a = pl.BlockSpec((_TB, _A), rows)
    row_spec_1 = pl.BlockSpec((_TB, 1), rows)

    action2d, logp, values2d = pl.pallas_call(
        _body,
        grid=grid,
        in_specs=[
            row_spec_s,
            row_spec_s,
            row_spec_a,
            row_spec_1,
            row_spec_a,
            pl.BlockSpec((_S, _H), whole),
            pl.BlockSpec((1, _H), whole),
            pl.BlockSpec((_H, _A), whole),
            pl.BlockSpec((1, _A), whole),
            pl.BlockSpec((_S, _H), whole),
            pl.BlockSpec((1, _H), whole),
            pl.BlockSpec((_H, 1), whole),
            pl.BlockSpec((1, 1), whole),
        ],
        out_specs=[row_spec_1, row_spec_a, row_spec_1],
        out_shape=[
            jax.ShapeDtypeStruct((_B, 1), jnp.int32),
            jax.ShapeDtypeStruct((_B, _A), jnp.float32),
            jax.ShapeDtypeStruct((_B, 1), jnp.float32),
        ],
        compiler_params=pltpu.CompilerParams(
            dimension_semantics=("parallel",),
        ),
    )(s, perfect_s, legal_actions, greedy.reshape(_B, 1), gum,
      W1, b1.reshape(1, _H), W2, b2.reshape(1, _A),
      V1, Vb1.reshape(1, _H), V2, Vb2.reshape(1, 1))

    return (action2d.reshape(_B), logp, values2d.reshape(_B))
